# baseline (speedup 1.0000x reference)
import functools

import jax
import jax.numpy as jnp
from jax import lax
from jax.experimental import pallas as pl
from jax.experimental.pallas import tpu as pltpu
from jax.experimental.pallas import tpu_sc as plsc

NC = 2
NS = 16
NW = NC * NS
B = 16384
D = 64
BPW = B // NW
CHUNK = 128
NCHUNK = BPW // CHUNK

_mesh = plsc.VectorSubcoreMesh(core_axis_name="c", subcore_axis_name="s")


@functools.partial(
    pl.kernel,
    out_type=(
        jax.ShapeDtypeStruct((B, D), jnp.float32),
        jax.ShapeDtypeStruct((B, D), jnp.float32),
        jax.ShapeDtypeStruct((B, D), jnp.float32),
    ),
    mesh=_mesh,
    compiler_params=pltpu.CompilerParams(use_tc_tiling_on_sc=False),
    scratch_types=[
        pltpu.VMEM((NCHUNK, CHUNK), jnp.int32),
        pltpu.VMEM((NCHUNK, CHUNK), jnp.int32),
        pltpu.VMEM((NCHUNK, CHUNK), jnp.int32),
        pltpu.VMEM((BPW, D), jnp.float32),
        pltpu.VMEM((BPW, D), jnp.float32),
        pltpu.VMEM((BPW, D), jnp.float32),
        pltpu.SemaphoreType.DMA,
        pltpu.SemaphoreType.DMA,
        pltpu.SemaphoreType.DMA,
        pltpu.SemaphoreType.DMA,
    ],
)
def _gather3(eu, ei, et, iu, ii, it, ou, oi, ot,
             idx_u, idx_i, idx_t, rows_u, rows_i, rows_t,
             sem_u, sem_i, sem_t, sem_out):
    wid = lax.axis_index("s") * NC + lax.axis_index("c")
    base = wid * BPW

    pltpu.sync_copy(iu.at[wid], idx_u)
    pltpu.sync_copy(ii.at[wid], idx_i)
    pltpu.sync_copy(it.at[wid], idx_t)

    gathers = []
    for tab, idx_v, rows_v, sem in (
        (eu, idx_u, rows_u, sem_u),
        (ei, idx_i, rows_i, sem_i),
        (et, idx_t, rows_t, sem_t),
    ):
        copies = []
        for j in range(NCHUNK):
            copies.append(pltpu.async_copy(
                tab.at[idx_v.at[j]],
                rows_v.at[pl.ds(j * CHUNK, CHUNK)],
                sem,
            ))
        gathers.append(copies)

    writes = []
    for copies, rows_v, out_hbm in (
        (gathers[0], rows_u, ou),
        (gathers[1], rows_i, oi),
        (gathers[2], rows_t, ot),
    ):
        for c in copies:
            c.wait()
        writes.append(
            pltpu.async_copy(rows_v, out_hbm.at[pl.ds(base, BPW)], sem_out)
        )
    for w in writes:
        w.wait()


def kernel(emb_user, emb_item, emb_tag, indices_user, indices_item, indices_tag):
    iu = indices_user.astype(jnp.int32).reshape(NW, NCHUNK, CHUNK)
    ii = indices_item.astype(jnp.int32).reshape(NW, NCHUNK, CHUNK)
    it = indices_tag.astype(jnp.int32).reshape(NW, NCHUNK, CHUNK)
    return _gather3(emb_user, emb_item, emb_tag, iu, ii, it)


# speedup vs baseline: 1.5523x; 1.5523x over previous
import functools

import jax
import jax.numpy as jnp
from jax import lax
from jax.experimental import pallas as pl
from jax.experimental.pallas import tpu as pltpu
from jax.experimental.pallas import tpu_sc as plsc

NC = 2
NS = 16
NW = NC * NS
B = 16384
D = 64
BPW = B // NW
HALF = BPW // 2

_mesh = plsc.VectorSubcoreMesh(core_axis_name="c", subcore_axis_name="s")


@functools.partial(
    pl.kernel,
    out_type=(
        jax.ShapeDtypeStruct((B, D), jnp.float32),
        jax.ShapeDtypeStruct((B, D), jnp.float32),
        jax.ShapeDtypeStruct((B, D), jnp.float32),
    ),
    mesh=_mesh,
    scratch_types=[
        pltpu.VMEM((BPW,), jnp.int32),
        pltpu.VMEM((BPW,), jnp.int32),
        pltpu.VMEM((BPW,), jnp.int32),
        pltpu.VMEM((HALF, D), jnp.float32),
        pltpu.VMEM((HALF, D), jnp.float32),
        pltpu.SemaphoreType.DMA,
        pltpu.SemaphoreType.DMA,
        pltpu.SemaphoreType.DMA,
        pltpu.SemaphoreType.DMA,
    ],
)
def _gather3(eu, ei, et, iu, ii, it, ou, oi, ot,
             idx_u, idx_i, idx_t, rows_a, rows_b,
             sem_ga, sem_gb, sem_wa, sem_wb):
    wid = lax.axis_index("s") * NC + lax.axis_index("c")
    base = wid * BPW

    pltpu.sync_copy(iu.at[pl.ds(base, BPW)], idx_u)
    pltpu.sync_copy(ii.at[pl.ds(base, BPW)], idx_i)
    pltpu.sync_copy(it.at[pl.ds(base, BPW)], idx_t)

    bufs = (rows_a, rows_b)
    gsems = (sem_ga, sem_gb)
    wsems = (sem_wa, sem_wb)

    def fire(tab, idx_v, off, rows_v, sem):
        def body(k, carry):
            rv = idx_v[pl.ds(off + k * 16, 16)]
            for l in range(16):
                r = rv[l]
                pltpu.async_copy(
                    tab.at[pl.ds(r, 1), :],
                    rows_v.at[pl.ds(k * 16 + l, 1), :],
                    sem,
                )
            return carry
        lax.fori_loop(0, HALF // 16, body, 0)

    def drain(tab, rows_v, sem):
        pltpu.make_async_copy(tab.at[pl.ds(0, HALF), :], rows_v, sem).wait()

    phases = [
        (eu, idx_u, ou, 0), (eu, idx_u, ou, 1),
        (ei, idx_i, oi, 0), (ei, idx_i, oi, 1),
        (et, idx_t, ot, 0), (et, idx_t, ot, 1),
    ]
    writes = [None, None]
    prev = [None]

    for p, (tab, idx_v, out_hbm, half) in enumerate(phases):
        b = p % 2
        if writes[b] is not None:
            writes[b].wait()
        fire(tab, idx_v, half * HALF, bufs[b], gsems[b])
        if prev[0] is not None:
            ptab, pout, phalf, pb = prev[0]
            drain(ptab, bufs[pb], gsems[pb])
            writes[pb] = pltpu.async_copy(
                bufs[pb], pout.at[pl.ds(base + phalf * HALF, HALF)], wsems[pb]
            )
        prev[0] = (tab, out_hbm, half, b)

    ptab, pout, phalf, pb = prev[0]
    drain(ptab, bufs[pb], gsems[pb])
    writes[pb] = pltpu.async_copy(
        bufs[pb], pout.at[pl.ds(base + phalf * HALF, HALF)], wsems[pb]
    )
    writes[0].wait()
    writes[1].wait()


def kernel(emb_user, emb_item, emb_tag, indices_user, indices_item, indices_tag):
    return _gather3(emb_user, emb_item, emb_tag,
                    indices_user.astype(jnp.int32),
                    indices_item.astype(jnp.int32),
                    indices_tag.astype(jnp.int32))
